# baseline (speedup 1.0000x reference)
import functools
import math

import jax
import jax.numpy as jnp
from jax import lax
from jax.experimental import pallas as pl
from jax.experimental.pallas import tpu as pltpu
from jax.experimental.pallas import tpu_sc as plsc

_EMB_DIM = 128
_D_MODEL = 512
_N_ROWS = 50 * 1024

_NC = 2
_NS = 16
_NW = _NC * _NS
_ROWS_PER_W = _N_ROWS // _NW
_CHUNK = 80
_NCHUNK = _ROWS_PER_W // _CHUNK


def _sc_gather_body(table, idx_hbm, out_hbm, idx_v, buf0, buf1,
                    gsem0, gsem1, osem0, osem1):
    wid = lax.axis_index("s") * _NC + lax.axis_index("c")
    pltpu.sync_copy(idx_hbm.at[wid], idx_v)
    base = wid * _ROWS_PER_W
    bufs = (buf0, buf1)
    gsems = (gsem0, gsem1)
    osems = (osem0, osem1)
    out_copies = [None, None]
    for j in range(_NCHUNK):
        p = j % 2
        if out_copies[p] is not None:
            out_copies[p].wait()
        g = pltpu.async_copy(table.at[idx_v.at[j]], bufs[p], gsems[p])
        g.wait()
        out_copies[p] = pltpu.async_copy(
            bufs[p], out_hbm.at[pl.ds(base + j * _CHUNK, _CHUNK)], osems[p])
    out_copies[0].wait()
    out_copies[1].wait()


@jax.jit
def _sc_gather(table, idx2):
    mesh = plsc.VectorSubcoreMesh(
        core_axis_name="c", subcore_axis_name="s",
        num_cores=_NC, num_subcores=_NS)
    return pl.kernel(
        _sc_gather_body,
        out_type=jax.ShapeDtypeStruct((_N_ROWS, _EMB_DIM), jnp.float32),
        mesh=mesh,
        scratch_types=[
            pltpu.VMEM((_NCHUNK, _CHUNK), jnp.int32),
            pltpu.VMEM((_CHUNK, _EMB_DIM), jnp.float32),
            pltpu.VMEM((_CHUNK, _EMB_DIM), jnp.float32),
            pltpu.SemaphoreType.DMA,
            pltpu.SemaphoreType.DMA,
            pltpu.SemaphoreType.DMA,
            pltpu.SemaphoreType.DMA,
        ],
    )(table, idx2)


def _mm_body(x_ref, w_ref, b_ref, o_ref):
    o_ref[...] = jnp.dot(
        x_ref[...], w_ref[...],
        preferred_element_type=jnp.float32) + b_ref[...]


_BM = 1024


@jax.jit
def _tc_matmul(x, w, b2):
    grid = (x.shape[0] // _BM,)
    return pl.pallas_call(
        _mm_body,
        grid=grid,
        in_specs=[
            pl.BlockSpec((_BM, _EMB_DIM), lambda i: (i, 0)),
            pl.BlockSpec((_EMB_DIM, _D_MODEL), lambda i: (0, 0)),
            pl.BlockSpec((1, _D_MODEL), lambda i: (0, 0)),
        ],
        out_specs=pl.BlockSpec((_BM, _D_MODEL), lambda i: (i, 0)),
        out_shape=jax.ShapeDtypeStruct((x.shape[0], _D_MODEL), jnp.float32),
        compiler_params=pltpu.CompilerParams(
            dimension_semantics=("parallel",)),
    )(x, w, b2)


def kernel(persona, persona_pad_mask, emb_table, W, b):
    del persona_pad_mask
    seq, batch = persona.shape
    idx2 = persona.reshape(_NW, _NCHUNK, _CHUNK)
    gathered = _sc_gather(emb_table, idx2)
    w2 = W.T * jnp.float32(math.sqrt(_EMB_DIM))
    out = _tc_matmul(gathered, w2, b.reshape(1, _D_MODEL))
    return out.reshape(seq, batch, _D_MODEL)


# speedup vs baseline: 1.1485x; 1.1485x over previous
import functools
import math

import jax
import jax.numpy as jnp
from jax import lax
from jax.experimental import pallas as pl
from jax.experimental.pallas import tpu as pltpu
from jax.experimental.pallas import tpu_sc as plsc

_EMB_DIM = 128
_D_MODEL = 512
_N_ROWS = 50 * 1024

_NC = 2
_NS = 16
_NW = _NC * _NS
_ROWS_PER_W = _N_ROWS // _NW
_CHUNK = 80
_NCHUNK = _ROWS_PER_W // _CHUNK


def _sc_gather_body(table, idx_hbm, out_hbm, idx_v, buf0, buf1,
                    gsem0, gsem1, osem0, osem1):
    wid = lax.axis_index("s") * _NC + lax.axis_index("c")
    pltpu.sync_copy(idx_hbm.at[wid], idx_v)
    base = wid * _ROWS_PER_W
    bufs = (buf0, buf1)
    gsems = (gsem0, gsem1)
    osems = (osem0, osem1)
    out_copies = [None, None]
    for j in range(_NCHUNK):
        p = j % 2
        if out_copies[p] is not None:
            out_copies[p].wait()
        g = pltpu.async_copy(table.at[idx_v.at[j]], bufs[p], gsems[p])
        g.wait()
        out_copies[p] = pltpu.async_copy(
            bufs[p], out_hbm.at[pl.ds(base + j * _CHUNK, _CHUNK)], osems[p])
    out_copies[0].wait()
    out_copies[1].wait()


@jax.jit
def _sc_gather(table, idx2):
    mesh = plsc.VectorSubcoreMesh(
        core_axis_name="c", subcore_axis_name="s",
        num_cores=_NC, num_subcores=_NS)
    return pl.kernel(
        _sc_gather_body,
        out_type=jax.ShapeDtypeStruct((_N_ROWS, _EMB_DIM), jnp.float32),
        mesh=mesh,
        scratch_types=[
            pltpu.VMEM((_NCHUNK, _CHUNK), jnp.int32),
            pltpu.VMEM((_CHUNK, _EMB_DIM), jnp.float32),
            pltpu.VMEM((_CHUNK, _EMB_DIM), jnp.float32),
            pltpu.SemaphoreType.DMA,
            pltpu.SemaphoreType.DMA,
            pltpu.SemaphoreType.DMA,
            pltpu.SemaphoreType.DMA,
        ],
    )(table, idx2)


def _mm_body(x_ref, w_ref, b_ref, o_ref):
    o_ref[...] = jnp.dot(
        x_ref[...], w_ref[...],
        preferred_element_type=jnp.float32) + b_ref[...]


_BM = 2048


@jax.jit
def _tc_matmul(x, w, b2):
    grid = (x.shape[0] // _BM,)
    return pl.pallas_call(
        _mm_body,
        grid=grid,
        in_specs=[
            pl.BlockSpec((_BM, _EMB_DIM), lambda i: (i, 0)),
            pl.BlockSpec((_EMB_DIM, _D_MODEL), lambda i: (0, 0)),
            pl.BlockSpec((1, _D_MODEL), lambda i: (0, 0)),
        ],
        out_specs=pl.BlockSpec((_BM, _D_MODEL), lambda i: (i, 0)),
        out_shape=jax.ShapeDtypeStruct((x.shape[0], _D_MODEL), jnp.float32),
        compiler_params=pltpu.CompilerParams(
            dimension_semantics=("parallel",)),
    )(x, w, b2)


def kernel(persona, persona_pad_mask, emb_table, W, b):
    del persona_pad_mask
    seq, batch = persona.shape
    idx2 = persona.reshape(_NW, _NCHUNK, _CHUNK)
    gathered = _sc_gather(emb_table, idx2)
    w2 = W.T * jnp.float32(math.sqrt(_EMB_DIM))
    out = _tc_matmul(gathered, w2, b.reshape(1, _D_MODEL))
    return out.reshape(seq, batch, _D_MODEL)


# speedup vs baseline: 1.2282x; 1.0694x over previous
import functools
import math

import jax
import jax.numpy as jnp
from jax import lax
from jax.experimental import pallas as pl
from jax.experimental.pallas import tpu as pltpu
from jax.experimental.pallas import tpu_sc as plsc

_EMB_DIM = 128
_D_MODEL = 512
_N_ROWS = 50 * 1024

_NC = 2
_NS = 16
_NW = _NC * _NS
_ROWS_PER_W = _N_ROWS // _NW
_CHUNK = 80
_NCHUNK = _ROWS_PER_W // _CHUNK


def _sc_gather_body(table, idx_hbm, out_hbm, idx_v, buf0, buf1,
                    gsem0, gsem1, osem0, osem1):
    wid = lax.axis_index("s") * _NC + lax.axis_index("c")
    pltpu.sync_copy(idx_hbm.at[wid], idx_v)
    base = wid * _ROWS_PER_W
    bufs = (buf0, buf1)
    gsems = (gsem0, gsem1)
    osems = (osem0, osem1)
    out_copies = [None, None]
    for j in range(_NCHUNK):
        p = j % 2
        if out_copies[p] is not None:
            out_copies[p].wait()
        g = pltpu.async_copy(table.at[idx_v.at[j]], bufs[p], gsems[p])
        g.wait()
        out_copies[p] = pltpu.async_copy(
            bufs[p], out_hbm.at[pl.ds(base + j * _CHUNK, _CHUNK)], osems[p])
    out_copies[0].wait()
    out_copies[1].wait()


@jax.jit
def _sc_gather(table, idx2):
    mesh = plsc.VectorSubcoreMesh(
        core_axis_name="c", subcore_axis_name="s",
        num_cores=_NC, num_subcores=_NS)
    return pl.kernel(
        _sc_gather_body,
        out_type=jax.ShapeDtypeStruct((_N_ROWS, _EMB_DIM), jnp.float32),
        mesh=mesh,
        scratch_types=[
            pltpu.VMEM((_NCHUNK, _CHUNK), jnp.int32),
            pltpu.VMEM((_CHUNK, _EMB_DIM), jnp.float32),
            pltpu.VMEM((_CHUNK, _EMB_DIM), jnp.float32),
            pltpu.SemaphoreType.DMA,
            pltpu.SemaphoreType.DMA,
            pltpu.SemaphoreType.DMA,
            pltpu.SemaphoreType.DMA,
        ],
    )(table, idx2)


def _mm_body(x_ref, w_ref, b_ref, o_ref):
    o_ref[...] = jnp.dot(
        x_ref[...], w_ref[...],
        preferred_element_type=jnp.float32) + b_ref[...]


_BM = 4096


@jax.jit
def _tc_matmul(x, w, b2):
    grid = (x.shape[0] // _BM,)
    return pl.pallas_call(
        _mm_body,
        grid=grid,
        in_specs=[
            pl.BlockSpec((_BM, _EMB_DIM), lambda i: (i, 0)),
            pl.BlockSpec((_EMB_DIM, _D_MODEL), lambda i: (0, 0)),
            pl.BlockSpec((1, _D_MODEL), lambda i: (0, 0)),
        ],
        out_specs=pl.BlockSpec((_BM, _D_MODEL), lambda i: (i, 0)),
        out_shape=jax.ShapeDtypeStruct((x.shape[0], _D_MODEL), jnp.float32),
        compiler_params=pltpu.CompilerParams(
            dimension_semantics=("parallel",)),
    )(x, w, b2)


def kernel(persona, persona_pad_mask, emb_table, W, b):
    del persona_pad_mask
    seq, batch = persona.shape
    idx2 = persona.reshape(_NW, _NCHUNK, _CHUNK)
    gathered = _sc_gather(emb_table, idx2)
    w2 = W.T * jnp.float32(math.sqrt(_EMB_DIM))
    out = _tc_matmul(gathered, w2, b.reshape(1, _D_MODEL))
    return out.reshape(seq, batch, _D_MODEL)


# speedup vs baseline: 1.2458x; 1.0143x over previous
import functools
import math

import jax
import jax.numpy as jnp
from jax import lax
from jax.experimental import pallas as pl
from jax.experimental.pallas import tpu as pltpu
from jax.experimental.pallas import tpu_sc as plsc

_EMB_DIM = 128
_D_MODEL = 512
_N_ROWS = 50 * 1024

_NC = 2
_NS = 16
_NW = _NC * _NS
_ROWS_PER_W = _N_ROWS // _NW
_CHUNK = 80
_NCHUNK = _ROWS_PER_W // _CHUNK


def _sc_gather_body(table, idx_hbm, out_hbm, idx_v, buf0, buf1,
                    gsem0, gsem1, osem0, osem1):
    wid = lax.axis_index("s") * _NC + lax.axis_index("c")
    pltpu.sync_copy(idx_hbm.at[wid], idx_v)
    base = wid * _ROWS_PER_W
    bufs = (buf0, buf1)
    gsems = (gsem0, gsem1)
    osems = (osem0, osem1)
    out_copies = [None, None]
    for j in range(_NCHUNK):
        p = j % 2
        if out_copies[p] is not None:
            out_copies[p].wait()
        g = pltpu.async_copy(table.at[idx_v.at[j]], bufs[p], gsems[p])
        g.wait()
        out_copies[p] = pltpu.async_copy(
            bufs[p], out_hbm.at[pl.ds(base + j * _CHUNK, _CHUNK)], osems[p])
    out_copies[0].wait()
    out_copies[1].wait()


@jax.jit
def _sc_gather(table, idx2):
    mesh = plsc.VectorSubcoreMesh(
        core_axis_name="c", subcore_axis_name="s",
        num_cores=_NC, num_subcores=_NS)
    return pl.kernel(
        _sc_gather_body,
        out_type=jax.ShapeDtypeStruct((_N_ROWS, _EMB_DIM), jnp.float32),
        mesh=mesh,
        scratch_types=[
            pltpu.VMEM((_NCHUNK, _CHUNK), jnp.int32),
            pltpu.VMEM((_CHUNK, _EMB_DIM), jnp.float32),
            pltpu.VMEM((_CHUNK, _EMB_DIM), jnp.float32),
            pltpu.SemaphoreType.DMA,
            pltpu.SemaphoreType.DMA,
            pltpu.SemaphoreType.DMA,
            pltpu.SemaphoreType.DMA,
        ],
    )(table, idx2)


def _mm_body(x_ref, w_ref, b_ref, o_ref):
    o_ref[...] = jnp.dot(
        x_ref[...], w_ref[...],
        preferred_element_type=jnp.float32) + b_ref[...]


_BM = 8192


@jax.jit
def _tc_matmul(x, w, b2):
    grid = (x.shape[0] // _BM,)
    return pl.pallas_call(
        _mm_body,
        grid=grid,
        in_specs=[
            pl.BlockSpec((_BM, _EMB_DIM), lambda i: (i, 0)),
            pl.BlockSpec((_EMB_DIM, _D_MODEL), lambda i: (0, 0)),
            pl.BlockSpec((1, _D_MODEL), lambda i: (0, 0)),
        ],
        out_specs=pl.BlockSpec((_BM, _D_MODEL), lambda i: (i, 0)),
        out_shape=jax.ShapeDtypeStruct((x.shape[0], _D_MODEL), jnp.float32),
        compiler_params=pltpu.CompilerParams(
            dimension_semantics=("parallel",)),
    )(x, w, b2)


def kernel(persona, persona_pad_mask, emb_table, W, b):
    del persona_pad_mask
    seq, batch = persona.shape
    idx2 = persona.reshape(_NW, _NCHUNK, _CHUNK)
    gathered = _sc_gather(emb_table, idx2)
    w2 = W.T * jnp.float32(math.sqrt(_EMB_DIM))
    out = _tc_matmul(gathered, w2, b.reshape(1, _D_MODEL))
    return out.reshape(seq, batch, _D_MODEL)


# speedup vs baseline: 1.2899x; 1.0354x over previous
import functools
import math

import jax
import jax.numpy as jnp
from jax import lax
from jax.experimental import pallas as pl
from jax.experimental.pallas import tpu as pltpu
from jax.experimental.pallas import tpu_sc as plsc

_EMB_DIM = 128
_D_MODEL = 512
_N_ROWS = 50 * 1024

_NC = 2
_NS = 16
_NW = _NC * _NS
_ROWS_PER_W = _N_ROWS // _NW
_CHUNK = 80
_NCHUNK = _ROWS_PER_W // _CHUNK


_RING = 6


def _sc_gather_body(table, idx_hbm, out_hbm, idx_v, *rest):
    bufs = rest[:_RING]
    gsems = rest[_RING:2 * _RING]
    osems = rest[2 * _RING:3 * _RING]
    wid = lax.axis_index("s") * _NC + lax.axis_index("c")
    pltpu.sync_copy(idx_hbm.at[wid], idx_v)
    base = wid * _ROWS_PER_W
    g = [None] * _RING
    o = [None] * _RING
    for j in range(_NCHUNK + _RING - 1):
        if j < _NCHUNK:
            p = j % _RING
            if o[p] is not None:
                o[p].wait()
            g[p] = pltpu.async_copy(table.at[idx_v.at[j]], bufs[p], gsems[p])
        d = j - (_RING - 1)
        if d >= 0:
            p = d % _RING
            g[p].wait()
            o[p] = pltpu.async_copy(
                bufs[p], out_hbm.at[pl.ds(base + d * _CHUNK, _CHUNK)],
                osems[p])
    for oc in o:
        if oc is not None:
            oc.wait()


@jax.jit
def _sc_gather(table, idx2):
    mesh = plsc.VectorSubcoreMesh(
        core_axis_name="c", subcore_axis_name="s",
        num_cores=_NC, num_subcores=_NS)
    return pl.kernel(
        _sc_gather_body,
        out_type=jax.ShapeDtypeStruct((_N_ROWS, _EMB_DIM), jnp.float32),
        mesh=mesh,
        scratch_types=(
            [pltpu.VMEM((_NCHUNK, _CHUNK), jnp.int32)]
            + [pltpu.VMEM((_CHUNK, _EMB_DIM), jnp.float32)] * _RING
            + [pltpu.SemaphoreType.DMA] * (2 * _RING)
        ),
    )(table, idx2)


def _mm_body(x_ref, w_ref, b_ref, o_ref):
    o_ref[...] = jnp.dot(
        x_ref[...], w_ref[...],
        preferred_element_type=jnp.float32) + b_ref[...]


_BM = 2048


@jax.jit
def _tc_matmul(x, w, b2):
    grid = (x.shape[0] // _BM,)
    return pl.pallas_call(
        _mm_body,
        grid=grid,
        in_specs=[
            pl.BlockSpec((_BM, _EMB_DIM), lambda i: (i, 0)),
            pl.BlockSpec((_EMB_DIM, _D_MODEL), lambda i: (0, 0)),
            pl.BlockSpec((1, _D_MODEL), lambda i: (0, 0)),
        ],
        out_specs=pl.BlockSpec((_BM, _D_MODEL), lambda i: (i, 0)),
        out_shape=jax.ShapeDtypeStruct((x.shape[0], _D_MODEL), jnp.float32),
        compiler_params=pltpu.CompilerParams(
            dimension_semantics=("parallel",)),
    )(x, w, b2)


def kernel(persona, persona_pad_mask, emb_table, W, b):
    del persona_pad_mask
    seq, batch = persona.shape
    idx2 = persona.reshape(_NW, _NCHUNK, _CHUNK)
    gathered = _sc_gather(emb_table, idx2)
    w2 = W.T * jnp.float32(math.sqrt(_EMB_DIM))
    out = _tc_matmul(gathered, w2, b.reshape(1, _D_MODEL))
    return out.reshape(seq, batch, _D_MODEL)


# speedup vs baseline: 1.3739x; 1.0651x over previous
import functools
import math

import jax
import jax.numpy as jnp
from jax import lax
from jax.experimental import pallas as pl
from jax.experimental.pallas import tpu as pltpu
from jax.experimental.pallas import tpu_sc as plsc

_EMB_DIM = 128
_D_MODEL = 512
_N_ROWS = 50 * 1024

_NC = 2
_NS = 16
_NW = _NC * _NS
_ROWS_PER_W = _N_ROWS // _NW
_CHUNK = 80
_NCHUNK = _ROWS_PER_W // _CHUNK


_RING = 6


def _sc_gather_body(table, idx_hbm, out_hbm, idx_v, *rest):
    bufs = rest[:_RING]
    gsems = rest[_RING:2 * _RING]
    osems = rest[2 * _RING:3 * _RING]
    wid = lax.axis_index("s") * _NC + lax.axis_index("c")
    pltpu.sync_copy(idx_hbm.at[wid], idx_v)
    base = wid * _ROWS_PER_W
    g = [None] * _RING
    o = [None] * _RING
    for j in range(_NCHUNK + _RING - 1):
        if j < _NCHUNK:
            p = j % _RING
            if o[p] is not None:
                o[p].wait()
            g[p] = pltpu.async_copy(table.at[idx_v.at[j]], bufs[p], gsems[p])
        d = j - (_RING - 1)
        if d >= 0:
            p = d % _RING
            g[p].wait()
            o[p] = pltpu.async_copy(
                bufs[p], out_hbm.at[pl.ds(base + d * _CHUNK, _CHUNK)],
                osems[p])
    for oc in o:
        if oc is not None:
            oc.wait()


@jax.jit
def _sc_gather(table, idx2):
    mesh = plsc.VectorSubcoreMesh(
        core_axis_name="c", subcore_axis_name="s",
        num_cores=_NC, num_subcores=_NS)
    return pl.kernel(
        _sc_gather_body,
        out_type=jax.ShapeDtypeStruct((_N_ROWS, _EMB_DIM), jnp.float32),
        mesh=mesh,
        scratch_types=(
            [pltpu.VMEM((_NCHUNK, _CHUNK), jnp.int32)]
            + [pltpu.VMEM((_CHUNK, _EMB_DIM), jnp.float32)] * _RING
            + [pltpu.SemaphoreType.DMA] * (2 * _RING)
        ),
    )(table, idx2)


def _mm_body(x_ref, w_ref, b_ref, o_ref):
    o_ref[...] = jnp.dot(
        x_ref[...], w_ref[...],
        preferred_element_type=jnp.float32) + b_ref[...]


_BM = 6400


@jax.jit
def _tc_matmul(x, w, b2):
    grid = (x.shape[0] // _BM,)
    return pl.pallas_call(
        _mm_body,
        grid=grid,
        in_specs=[
            pl.BlockSpec((_BM, _EMB_DIM), lambda i: (i, 0)),
            pl.BlockSpec((_EMB_DIM, _D_MODEL), lambda i: (0, 0)),
            pl.BlockSpec((1, _D_MODEL), lambda i: (0, 0)),
        ],
        out_specs=pl.BlockSpec((_BM, _D_MODEL), lambda i: (i, 0)),
        out_shape=jax.ShapeDtypeStruct((x.shape[0], _D_MODEL), jnp.float32),
        compiler_params=pltpu.CompilerParams(
            dimension_semantics=("parallel",)),
    )(x, w, b2)


def kernel(persona, persona_pad_mask, emb_table, W, b):
    del persona_pad_mask
    seq, batch = persona.shape
    idx2 = persona.reshape(_NW, _NCHUNK, _CHUNK)
    gathered = _sc_gather(emb_table, idx2)
    w2 = W.T * jnp.float32(math.sqrt(_EMB_DIM))
    out = _tc_matmul(gathered, w2, b.reshape(1, _D_MODEL))
    return out.reshape(seq, batch, _D_MODEL)
